# baseline (speedup 1.0000x reference)
import functools

import jax
import jax.numpy as jnp
from jax import lax
from jax.experimental import pallas as pl
from jax.experimental.pallas import tpu as pltpu
from jax.experimental.pallas import tpu_sc as plsc

NUM_PATCHES = 8192
PROJ_DIM = 1024


CHUNK_ROWS = 32
NBUF = 3


@functools.lru_cache(maxsize=None)
def _make_copy_kernel():
    info = plsc.get_sparse_core_info()
    nw = info.num_cores * info.num_subcores
    rows_per_w = NUM_PATCHES // nw
    n_ch = rows_per_w // CHUNK_ROWS

    mesh = plsc.VectorSubcoreMesh(core_axis_name="c", subcore_axis_name="s")
    ns = info.num_subcores

    @functools.partial(
        pl.kernel,
        out_type=jax.ShapeDtypeStruct((NUM_PATCHES, PROJ_DIM), jnp.float32),
        mesh=mesh,
        scratch_types=(
            [pltpu.VMEM_SHARED((ns, NBUF * CHUNK_ROWS, PROJ_DIM), jnp.float32)]
            + [pltpu.SemaphoreType.DMA] * (2 * NBUF)
        ),
    )
    def copy_rows(table_hbm, out_hbm, shared, *sems):
        sin = sems[:NBUF]
        sout = sems[NBUF:]
        sid = lax.axis_index("s")
        wid = sid * info.num_cores + lax.axis_index("c")
        base = wid * rows_per_w

        def in_copy(i):
            b = i % NBUF
            return pltpu.async_copy(
                table_hbm.at[pl.ds(base + i * CHUNK_ROWS, CHUNK_ROWS)],
                shared.at[sid, pl.ds(b * CHUNK_ROWS, CHUNK_ROWS)], sin[b])

        def out_copy(i):
            b = i % NBUF
            return pltpu.async_copy(
                shared.at[sid, pl.ds(b * CHUNK_ROWS, CHUNK_ROWS)],
                out_hbm.at[pl.ds(base + i * CHUNK_ROWS, CHUNK_ROWS)],
                sout[b])

        h_in = [None] * n_ch
        h_out = [None] * n_ch
        h_in[0] = in_copy(0)
        for i in range(n_ch):
            if i + 1 < n_ch:
                if i + 1 - NBUF >= 0:
                    h_out[i + 1 - NBUF].wait()
                h_in[i + 1] = in_copy(i + 1)
            h_in[i].wait()
            h_out[i] = out_copy(i)
        for j in range(max(0, n_ch - NBUF), n_ch):
            h_out[j].wait()

    return copy_rows


def kernel(tokens, table):
    del tokens
    out = _make_copy_kernel()(table)
    return out[None]


# speedup vs baseline: 1.0037x; 1.0037x over previous
import functools

import jax
import jax.numpy as jnp
from jax import lax
from jax.experimental import pallas as pl
from jax.experimental.pallas import tpu as pltpu
from jax.experimental.pallas import tpu_sc as plsc

NUM_PATCHES = 8192
PROJ_DIM = 1024


CHUNK_ROWS = 512
NBUF = 3


@functools.lru_cache(maxsize=None)
def _make_copy_kernel():
    info = plsc.get_sparse_core_info()
    nc = info.num_cores
    rows_per_c = NUM_PATCHES // nc
    n_ch = rows_per_c // CHUNK_ROWS

    mesh = plsc.ScalarSubcoreMesh(axis_name="c", num_cores=nc)

    @functools.partial(
        pl.kernel,
        out_type=jax.ShapeDtypeStruct((NUM_PATCHES, PROJ_DIM), jnp.float32),
        mesh=mesh,
        scratch_types=(
            [pltpu.VMEM_SHARED((NBUF * CHUNK_ROWS, PROJ_DIM), jnp.float32)]
            + [pltpu.SemaphoreType.DMA] * (2 * NBUF)
        ),
    )
    def copy_rows(table_hbm, out_hbm, shared, *sems):
        sin = sems[:NBUF]
        sout = sems[NBUF:]
        base = lax.axis_index("c") * rows_per_c

        def in_copy(i):
            b = i % NBUF
            return pltpu.async_copy(
                table_hbm.at[pl.ds(base + i * CHUNK_ROWS, CHUNK_ROWS)],
                shared.at[pl.ds(b * CHUNK_ROWS, CHUNK_ROWS)], sin[b])

        def out_copy(i):
            b = i % NBUF
            return pltpu.async_copy(
                shared.at[pl.ds(b * CHUNK_ROWS, CHUNK_ROWS)],
                out_hbm.at[pl.ds(base + i * CHUNK_ROWS, CHUNK_ROWS)],
                sout[b])

        h_in = [None] * n_ch
        h_out = [None] * n_ch
        h_in[0] = in_copy(0)
        for i in range(n_ch):
            if i + 1 < n_ch:
                if i + 1 - NBUF >= 0:
                    h_out[i + 1 - NBUF].wait()
                h_in[i + 1] = in_copy(i + 1)
            h_in[i].wait()
            h_out[i] = out_copy(i)
        for j in range(max(0, n_ch - NBUF), n_ch):
            h_out[j].wait()

    return copy_rows


def kernel(tokens, table):
    del tokens
    out = _make_copy_kernel()(table)
    return out[None]


# speedup vs baseline: 1.7546x; 1.7480x over previous
import functools

import jax
import jax.numpy as jnp
from jax.experimental import pallas as pl

NUM_PATCHES = 8192
PROJ_DIM = 1024
BLOCK_ROWS = 512


def _copy_body(table_ref, out_ref):
    out_ref[...] = table_ref[...]


@functools.lru_cache(maxsize=None)
def _make_tc_copy():
    grid = (NUM_PATCHES // BLOCK_ROWS,)
    return pl.pallas_call(
        _copy_body,
        grid=grid,
        in_specs=[pl.BlockSpec((BLOCK_ROWS, PROJ_DIM), lambda i: (i, 0))],
        out_specs=pl.BlockSpec((BLOCK_ROWS, PROJ_DIM), lambda i: (i, 0)),
        out_shape=jax.ShapeDtypeStruct((NUM_PATCHES, PROJ_DIM), jnp.float32),
    )


def kernel(tokens, table):
    del tokens
    return _make_tc_copy()(table)[None]


# speedup vs baseline: 1.9230x; 1.0960x over previous
import functools

import jax
import jax.numpy as jnp
from jax.experimental import pallas as pl

NUM_PATCHES = 8192
PROJ_DIM = 1024
BLOCK_ROWS = 1024


def _copy_body(table_ref, out_ref):
    out_ref[...] = table_ref[...]


@functools.lru_cache(maxsize=None)
def _make_tc_copy():
    grid = (NUM_PATCHES // BLOCK_ROWS,)
    return pl.pallas_call(
        _copy_body,
        grid=grid,
        in_specs=[pl.BlockSpec((BLOCK_ROWS, PROJ_DIM), lambda i: (i, 0))],
        out_specs=pl.BlockSpec((BLOCK_ROWS, PROJ_DIM), lambda i: (i, 0)),
        out_shape=jax.ShapeDtypeStruct((NUM_PATCHES, PROJ_DIM), jnp.float32),
    )


def kernel(tokens, table):
    del tokens
    return _make_tc_copy()(table)[None]


# speedup vs baseline: 2.0943x; 1.0891x over previous
import functools

import jax
import jax.numpy as jnp
from jax.experimental import pallas as pl

NUM_PATCHES = 8192
PROJ_DIM = 1024
BLOCK_ROWS = 2048


def _copy_body(table_ref, out_ref):
    out_ref[...] = table_ref[...]


@functools.lru_cache(maxsize=None)
def _make_tc_copy():
    grid = (NUM_PATCHES // BLOCK_ROWS,)
    return pl.pallas_call(
        _copy_body,
        grid=grid,
        in_specs=[pl.BlockSpec((BLOCK_ROWS, PROJ_DIM), lambda i: (i, 0))],
        out_specs=pl.BlockSpec((BLOCK_ROWS, PROJ_DIM), lambda i: (i, 0)),
        out_shape=jax.ShapeDtypeStruct((NUM_PATCHES, PROJ_DIM), jnp.float32),
    )


def kernel(tokens, table):
    del tokens
    return _make_tc_copy()(table)[None]
